# baseline (speedup 1.0000x reference)
import jax
import jax.numpy as jnp
from jax import lax
from jax.experimental import pallas as pl
from jax.experimental.pallas import tpu as pltpu
from jax.experimental.pallas import tpu_sc as plsc

NUM_ROWS = 2600000
BATCH = 16384
NUM_FIELDS = 26

_info = plsc.get_sparse_core_info()
NC, NS, L = _info.num_cores, _info.num_subcores, _info.num_lanes
NW = NC * NS
B_PER_W = BATCH // NW
E_PER_W = B_PER_W * NUM_FIELDS
CHUNK = 128
N_DMA = E_PER_W // CHUNK
N_OUT_CHUNKS = B_PER_W // L


def _sc_kernel(x_hbm, table_hbm, out_hbm, idx2d_v, idx_v, vals_v, out_v, sem):
    wid = lax.axis_index("s") * NC + lax.axis_index("c")

    base_row = pl.multiple_of(wid * B_PER_W, B_PER_W)
    pltpu.sync_copy(x_hbm.at[pl.ds(base_row, B_PER_W)], idx2d_v)

    lane = lax.iota(jnp.int32, L)

    def repack(i, c):
        rows = i * L + lane
        for f in range(NUM_FIELDS):
            g = plsc.load_gather(idx2d_v, [rows, jnp.full((L,), f, jnp.int32)])
            idx_v[pl.ds(pl.multiple_of(i * (L * NUM_FIELDS) + f * L, L), L)] = g
        return c

    lax.fori_loop(0, N_OUT_CHUNKS, repack, 0)

    def fire(j, c):
        dst = vals_v.at[pl.ds(pl.multiple_of(j * CHUNK, CHUNK), CHUNK)]
        pltpu.async_copy(table_hbm.at[idx_v.at[pl.ds(pl.multiple_of(j * CHUNK, CHUNK), CHUNK)]], dst, sem)
        return c

    lax.fori_loop(0, N_DMA, fire, 0)

    def drain(j, c):
        dst = vals_v.at[pl.ds(pl.multiple_of(j * CHUNK, CHUNK), CHUNK)]
        pltpu.make_async_copy(table_hbm.at[idx_v.at[pl.ds(pl.multiple_of(j * CHUNK, CHUNK), CHUNK)]], dst, sem).wait()
        return c

    lax.fori_loop(0, N_DMA, drain, 0)

    def reduce_chunk(i, c):
        base = i * (L * NUM_FIELDS)
        acc = jnp.zeros((L,), jnp.float32)
        for f in range(NUM_FIELDS):
            acc = acc + vals_v[pl.ds(pl.multiple_of(base + f * L, L), L)]
        out_v[pl.ds(pl.multiple_of(i * L, L), L)] = acc
        return c

    lax.fori_loop(0, N_OUT_CHUNKS, reduce_chunk, 0)

    pltpu.sync_copy(out_v, out_hbm.at[pl.ds(base_row, B_PER_W)])


@jax.jit
def kernel(x, table):
    table_flat = table.reshape(NUM_ROWS)
    mesh = plsc.VectorSubcoreMesh(core_axis_name="c", subcore_axis_name="s")
    out = pl.kernel(
        _sc_kernel,
        mesh=mesh,
        compiler_params=pltpu.CompilerParams(needs_layout_passes=False),
        out_type=jax.ShapeDtypeStruct((BATCH,), jnp.float32),
        scratch_types=[
            pltpu.VMEM((B_PER_W, NUM_FIELDS), jnp.int32),
            pltpu.VMEM((E_PER_W,), jnp.int32),
            pltpu.VMEM((E_PER_W,), jnp.float32),
            pltpu.VMEM((B_PER_W,), jnp.float32),
            pltpu.SemaphoreType.DMA,
        ],
    )(x, table_flat)
    return out.reshape(BATCH, 1)


# speedup vs baseline: 1.0772x; 1.0772x over previous
import jax
import jax.numpy as jnp
from jax import lax
from jax.experimental import pallas as pl
from jax.experimental.pallas import tpu as pltpu
from jax.experimental.pallas import tpu_sc as plsc

NUM_ROWS = 2600000
BATCH = 16384
NUM_FIELDS = 26

_info = plsc.get_sparse_core_info()
NC, NS, L = _info.num_cores, _info.num_subcores, _info.num_lanes
NW = NC * NS
B_PER_W = BATCH // NW
E_PER_W = B_PER_W * NUM_FIELDS
CHUNK = 128
N_DMA = E_PER_W // CHUNK
N_OUT_CHUNKS = B_PER_W // L


def _repack_kernel(x_hbm, idx_out_hbm, idx2d_v, idx_v):
    wid = lax.axis_index("s") * NC + lax.axis_index("c")
    base_row = pl.multiple_of(wid * B_PER_W, B_PER_W)
    pltpu.sync_copy(x_hbm.at[pl.ds(base_row, B_PER_W)], idx2d_v)

    lane = lax.iota(jnp.int32, L)

    def repack(i, c):
        rows = i * L + lane
        for f in range(NUM_FIELDS):
            g = plsc.load_gather(idx2d_v, [rows, jnp.full((L,), f, jnp.int32)])
            idx_v[pl.ds(pl.multiple_of(i * (L * NUM_FIELDS) + f * L, L), L)] = g
        return c

    lax.fori_loop(0, N_OUT_CHUNKS, repack, 0)

    base_e = pl.multiple_of(wid * E_PER_W, E_PER_W)
    pltpu.sync_copy(idx_v, idx_out_hbm.at[pl.ds(base_e, E_PER_W)])


def _gather_kernel(idx_hbm, table_hbm, out_hbm, idx_v, vals_v, out_v, sem):
    wid = lax.axis_index("s") * NC + lax.axis_index("c")
    base_e = pl.multiple_of(wid * E_PER_W, E_PER_W)
    pltpu.sync_copy(idx_hbm.at[pl.ds(base_e, E_PER_W)], idx_v)

    def fire(j, c):
        sl = pl.ds(pl.multiple_of(j * CHUNK, CHUNK), CHUNK)
        pltpu.async_copy(table_hbm.at[idx_v.at[sl]], vals_v.at[sl], sem)
        return c

    lax.fori_loop(0, N_DMA, fire, 0)

    def drain(j, c):
        sl = pl.ds(pl.multiple_of(j * CHUNK, CHUNK), CHUNK)
        pltpu.make_async_copy(table_hbm.at[idx_v.at[sl]], vals_v.at[sl], sem).wait()
        return c

    lax.fori_loop(0, N_DMA, drain, 0)

    def reduce_chunk(i, c):
        base = i * (L * NUM_FIELDS)
        acc = jnp.zeros((L,), jnp.float32)
        for f in range(NUM_FIELDS):
            acc = acc + vals_v[pl.ds(pl.multiple_of(base + f * L, L), L)]
        out_v[pl.ds(pl.multiple_of(i * L, L), L)] = acc
        return c

    lax.fori_loop(0, N_OUT_CHUNKS, reduce_chunk, 0)

    base_row = pl.multiple_of(wid * B_PER_W, B_PER_W)
    pltpu.sync_copy(out_v, out_hbm.at[pl.ds(base_row, B_PER_W)])


@jax.jit
def kernel(x, table):
    mesh = plsc.VectorSubcoreMesh(core_axis_name="c", subcore_axis_name="s")
    idx_flat = pl.kernel(
        _repack_kernel,
        mesh=mesh,
        compiler_params=pltpu.CompilerParams(needs_layout_passes=False),
        out_type=jax.ShapeDtypeStruct((BATCH * NUM_FIELDS,), jnp.int32),
        scratch_types=[
            pltpu.VMEM((B_PER_W, NUM_FIELDS), jnp.int32),
            pltpu.VMEM((E_PER_W,), jnp.int32),
        ],
    )(x)
    table_flat = table.reshape(NUM_ROWS)
    out = pl.kernel(
        _gather_kernel,
        mesh=mesh,
        compiler_params=pltpu.CompilerParams(needs_layout_passes=False),
        out_type=jax.ShapeDtypeStruct((BATCH,), jnp.float32),
        scratch_types=[
            pltpu.VMEM((E_PER_W,), jnp.int32),
            pltpu.VMEM((E_PER_W,), jnp.float32),
            pltpu.VMEM((B_PER_W,), jnp.float32),
            pltpu.SemaphoreType.DMA,
        ],
    )(idx_flat, table_flat)
    return out.reshape(BATCH, 1)


# speedup vs baseline: 1.1280x; 1.0471x over previous
import jax
import jax.numpy as jnp
from jax import lax
from jax.experimental import pallas as pl
from jax.experimental.pallas import tpu as pltpu
from jax.experimental.pallas import tpu_sc as plsc

NUM_ROWS = 2600000
BATCH = 16384
NUM_FIELDS = 26

_info = plsc.get_sparse_core_info()
NC, NS, L = _info.num_cores, _info.num_subcores, _info.num_lanes
NW = NC * NS
B_PER_W = BATCH // NW
E_PER_W = B_PER_W * NUM_FIELDS
CHUNK = 128
N_DMA = E_PER_W // CHUNK
N_OUT_CHUNKS = B_PER_W // L
N_GRP = 8
DMA_PER_GRP = N_DMA // N_GRP
CHUNKS_PER_GRP = N_OUT_CHUNKS // N_GRP


def _repack_kernel(xt_hbm, idx_out_hbm, xt_v, idx_v):
    wid = lax.axis_index("s") * NC + lax.axis_index("c")
    base_row = pl.multiple_of(wid * B_PER_W, B_PER_W)
    pltpu.sync_copy(xt_hbm.at[:, pl.ds(base_row, B_PER_W)], xt_v)

    def repack(i, c):
        src = pl.ds(pl.multiple_of(i * L, L), L)
        for f in range(NUM_FIELDS):
            g = xt_v[f, src]
            idx_v[pl.ds(pl.multiple_of(i * (L * NUM_FIELDS) + f * L, L), L)] = g
        return c

    lax.fori_loop(0, N_OUT_CHUNKS, repack, 0)

    base_e = pl.multiple_of(wid * E_PER_W, E_PER_W)
    pltpu.sync_copy(idx_v, idx_out_hbm.at[pl.ds(base_e, E_PER_W)])


def _gather_kernel(idx_hbm, table_hbm, out_hbm, idx_v, vals_v, out_v, sems):
    wid = lax.axis_index("s") * NC + lax.axis_index("c")
    base_e = pl.multiple_of(wid * E_PER_W, E_PER_W)
    pltpu.sync_copy(idx_hbm.at[pl.ds(base_e, E_PER_W)], idx_v)

    def fire(j, c):
        sl = pl.ds(pl.multiple_of(j * CHUNK, CHUNK), CHUNK)
        pltpu.async_copy(table_hbm.at[idx_v.at[sl]], vals_v.at[sl], sems.at[j // DMA_PER_GRP])
        return c

    lax.fori_loop(0, N_DMA, fire, 0)

    def group(b, c):
        def drain(j, cc):
            sl = pl.ds(pl.multiple_of((b * DMA_PER_GRP + j) * CHUNK, CHUNK), CHUNK)
            pltpu.make_async_copy(table_hbm.at[idx_v.at[sl]], vals_v.at[sl], sems.at[b]).wait()
            return cc

        lax.fori_loop(0, DMA_PER_GRP, drain, 0)

        def reduce_chunk(i, cc):
            base = (b * CHUNKS_PER_GRP + i) * (L * NUM_FIELDS)
            acc = jnp.zeros((L,), jnp.float32)
            for f in range(NUM_FIELDS):
                acc = acc + vals_v[pl.ds(pl.multiple_of(base + f * L, L), L)]
            out_v[pl.ds(pl.multiple_of((b * CHUNKS_PER_GRP + i) * L, L), L)] = acc
            return cc

        lax.fori_loop(0, CHUNKS_PER_GRP, reduce_chunk, 0)
        return c

    lax.fori_loop(0, N_GRP, group, 0)

    base_row = pl.multiple_of(wid * B_PER_W, B_PER_W)
    pltpu.sync_copy(out_v, out_hbm.at[pl.ds(base_row, B_PER_W)])


@jax.jit
def kernel(x, table):
    mesh = plsc.VectorSubcoreMesh(core_axis_name="c", subcore_axis_name="s")
    idx_flat = pl.kernel(
        _repack_kernel,
        mesh=mesh,
        compiler_params=pltpu.CompilerParams(needs_layout_passes=False),
        out_type=jax.ShapeDtypeStruct((BATCH * NUM_FIELDS,), jnp.int32),
        scratch_types=[
            pltpu.VMEM((NUM_FIELDS, B_PER_W), jnp.int32),
            pltpu.VMEM((E_PER_W,), jnp.int32),
        ],
    )(x.T)
    table_flat = table.reshape(NUM_ROWS)
    out = pl.kernel(
        _gather_kernel,
        mesh=mesh,
        compiler_params=pltpu.CompilerParams(needs_layout_passes=False),
        out_type=jax.ShapeDtypeStruct((BATCH,), jnp.float32),
        scratch_types=[
            pltpu.VMEM((E_PER_W,), jnp.int32),
            pltpu.VMEM((E_PER_W,), jnp.float32),
            pltpu.VMEM((B_PER_W,), jnp.float32),
            pltpu.SemaphoreType.DMA((N_GRP,)),
        ],
    )(idx_flat, table_flat)
    return out.reshape(BATCH, 1)
